# Initial kernel scaffold; baseline (speedup 1.0000x reference)
import jax
import jax.numpy as jnp
from jax.experimental import pallas as pl


def kernel(x, batch, edge_index, W0, b0, W1, b1, W2, b2, gn0_w, gn0_b, gn0_s, gn1_w, gn1_b, gn1_s, gn2_w, gn2_b, gn2_s):
    raise NotImplementedError("write your pallas kernel here")


# speedup vs baseline: 21.2117x; 21.2117x over previous
import functools

import jax
import jax.numpy as jnp
from jax import lax
from jax.experimental import pallas as pl
from jax.experimental.pallas import tpu as pltpu
from jax.experimental.pallas import tpu_sc as plsc

N = 50000
T = 2
C = 3
E = 800000
EMB = 128
HID = 32
G = 32

R = 1024
N_PAD = 50176
NBLK = N_PAD // R
ROWS_PER_TILE = N_PAD // 16

E_PAD = 802816
ECHUNKS = E_PAD // 128
CH_PER_TILE = ECHUNKS // 16
CH_PER_WORKER = ECHUNKS // 32
IGRP = 56
ZROWS = 448

EPS = 1e-5

_MESH = dict(core_axis_name="c", subcore_axis_name="s", num_cores=2,
             num_subcores=16)


def _deg_body(dst_hbm, out_hbm, hist, idxb):
    c = lax.axis_index("c")
    s = lax.axis_index("s")
    wid = s * 2 + c
    zero16 = jnp.zeros((16,), jnp.float32)
    one16 = jnp.ones((16,), jnp.float32)

    def zloop(i, _):
        hist[pl.ds(i * 16, 16)] = zero16
        return 0
    lax.fori_loop(0, N_PAD // 16, zloop, 0)

    pltpu.sync_copy(dst_hbm.at[pl.ds(wid * CH_PER_WORKER, CH_PER_WORKER)],
                    idxb)

    def eloop(g, _):
        for k in range(8):
            idx = idxb[g, pl.ds(k * 16, 16)]
            plsc.addupdate_scatter(hist, [idx], one16)
        return 0
    lax.fori_loop(0, CH_PER_WORKER, eloop, 0)

    pltpu.sync_copy(hist, out_hbm.at[wid])


def _sc_degree(dstA):
    mesh = plsc.VectorSubcoreMesh(**_MESH)
    return pl.kernel(
        _deg_body,
        out_type=jax.ShapeDtypeStruct((32, N_PAD), jnp.float32),
        mesh=mesh,
        scratch_types=[
            pltpu.VMEM((N_PAD,), jnp.float32),
            pltpu.VMEM((CH_PER_WORKER, 128), jnp.int32),
        ],
        compiler_params=pltpu.CompilerParams(use_tc_tiling_on_sc=False, needs_layout_passes=False),
        name="sc_degree",
    )(dstA)


def _agg_body(tab_hbm, srcA_hbm, dstA_hbm, zrows_hbm, out_hbm,
              acc, srcb, dstb, rows, sem):
    c = lax.axis_index("c")
    s = lax.axis_index("s")

    def zloop(z, _):
        pltpu.sync_copy(zrows_hbm, acc.at[pl.ds(s * ROWS_PER_TILE + z * ZROWS,
                                                ZROWS)])
        return 0
    lax.fori_loop(0, ROWS_PER_TILE // ZROWS, zloop, 0)
    plsc.subcore_barrier()

    row0 = s * CH_PER_TILE

    def grp(g, _):
        pltpu.sync_copy(srcA_hbm.at[c, pl.ds(row0 + g * IGRP, IGRP)], srcb)
        pltpu.sync_copy(dstA_hbm.at[pl.ds(row0 + g * IGRP, IGRP)], dstb)

        def chunk(j, _):
            pltpu.async_copy(tab_hbm.at[srcb.at[j]], rows, sem).wait()
            pltpu.sync_copy(rows, acc.at[dstb.at[j]], add=True)
            return 0
        lax.fori_loop(0, IGRP, chunk, 0)
        return 0
    lax.fori_loop(0, CH_PER_TILE // IGRP, grp, 0)

    plsc.subcore_barrier()
    sl = pl.ds(s * ROWS_PER_TILE, ROWS_PER_TILE)
    pltpu.sync_copy(acc.at[sl], out_hbm.at[c, sl])


def _sc_aggregate(tab, srcA, dstA, zrows, d):
    mesh = plsc.VectorSubcoreMesh(**_MESH)
    return pl.kernel(
        _agg_body,
        out_type=jax.ShapeDtypeStruct((2, N_PAD, d), jnp.float32),
        mesh=mesh,
        scratch_types=[
            pltpu.VMEM_SHARED((N_PAD, d), jnp.float32),
            pltpu.VMEM((IGRP, 128), jnp.int32),
            pltpu.VMEM((IGRP, 128), jnp.int32),
            pltpu.VMEM((128, d), jnp.float32),
            pltpu.SemaphoreType.DMA,
        ],
        compiler_params=pltpu.CompilerParams(use_tc_tiling_on_sc=False, needs_layout_passes=False),
        name=f"sc_aggregate_d{d}",
    )(tab, srcA, dstA, zrows)


def _onehotT(batch_ref):
    bb = batch_ref[0]
    gi = lax.broadcasted_iota(jnp.int32, (G, 1), 0)
    return (bb == gi).astype(jnp.float32)


def _gelu(v):
    return 0.5 * v * (1.0 + lax.erf(v * 0.7071067811865476))


def _prep_body(degp_ref, x2_ref, batch_ref, dinv_ref, u0_ref, cnt_ref,
               cnt_acc):
    i = pl.program_id(0)
    deg = jnp.sum(degp_ref[...], axis=1, keepdims=True) + 1.0
    dinv = lax.rsqrt(deg)
    dinv_ref[...] = dinv
    xb = x2_ref[...]
    pad = jnp.zeros((R, 16 - C), jnp.float32)
    u0 = jnp.stack([jnp.concatenate([xb[:, 0:C] * dinv, pad], axis=1),
                    jnp.concatenate([xb[:, C:2 * C] * dinv, pad], axis=1)])
    u0_ref[...] = u0
    mT = _onehotT(batch_ref)

    @pl.when(i == 0)
    def _():
        cnt_acc[...] = jnp.zeros_like(cnt_acc)
    cnt_acc[...] += jnp.sum(mT, axis=1, keepdims=True)

    @pl.when(i == NBLK - 1)
    def _():
        cnt_ref[...] = jnp.maximum(cnt_acc[...], 1.0)


def _tc_prep(degpT, x2, batch3):
    grid = (NBLK,)
    return pl.pallas_call(
        _prep_body,
        grid=grid,
        in_specs=[
            pl.BlockSpec((R, 32), lambda i: (i, 0)),
            pl.BlockSpec((R, 2 * C), lambda i: (i, 0)),
            pl.BlockSpec((1, 1, R), lambda i: (i, 0, 0)),
        ],
        out_specs=[
            pl.BlockSpec((R, 1), lambda i: (i, 0)),
            pl.BlockSpec((2, R, 16), lambda i: (0, i, 0)),
            pl.BlockSpec((G, 1), lambda i: (0, 0)),
        ],
        out_shape=[
            jax.ShapeDtypeStruct((N_PAD, 1), jnp.float32),
            jax.ShapeDtypeStruct((2, N_PAD, 16), jnp.float32),
            jax.ShapeDtypeStruct((G, 1), jnp.float32),
        ],
        scratch_shapes=[pltpu.VMEM((G, 1), jnp.float32)],
        name="tc_prep",
    )(degpT, x2, batch3)


def _matstats_body(din, fout, agg_ref, u_ref, dinv_ref, batch_ref, w_ref,
                   b_ref, z_ref, s1_ref, s2_ref, s1_acc, s2_acc):
    i = pl.program_id(0)
    dinv = dinv_ref[...]
    w = w_ref[...]
    zs = []
    for t in range(2):
        a = agg_ref[t] + u_ref[t]
        a = a[:, :w.shape[0]]
        zs.append(lax.dot_general(a, w, (((1,), (0,)), ((), ())),
                                  preferred_element_type=jnp.float32,
                           precision=lax.Precision.HIGHEST))
    z = jnp.concatenate(zs, axis=1) * dinv + b_ref[...]
    z_ref[...] = z
    mT = _onehotT(batch_ref)

    @pl.when(i == 0)
    def _():
        s1_acc[...] = jnp.zeros_like(s1_acc)
        s2_acc[...] = jnp.zeros_like(s2_acc)
    s1_acc[...] += lax.dot_general(mT, z, (((1,), (0,)), ((), ())),
                                   preferred_element_type=jnp.float32,
                           precision=lax.Precision.HIGHEST)
    s2_acc[...] += lax.dot_general(mT, z * z, (((1,), (0,)), ((), ())),
                                   preferred_element_type=jnp.float32,
                           precision=lax.Precision.HIGHEST)

    @pl.when(i == NBLK - 1)
    def _():
        s1_ref[...] = s1_acc[...]
        s2_ref[...] = s2_acc[...]


def _tc_matstats(agg, u, dinv, batch3, w, b2, din, fout):
    grid = (NBLK,)
    f2 = 2 * fout
    return pl.pallas_call(
        functools.partial(_matstats_body, din, fout),
        grid=grid,
        in_specs=[
            pl.BlockSpec((2, R, din), lambda i: (0, i, 0)),
            pl.BlockSpec((2, R, din), lambda i: (0, i, 0)),
            pl.BlockSpec((R, 1), lambda i: (i, 0)),
            pl.BlockSpec((1, 1, R), lambda i: (i, 0, 0)),
            pl.BlockSpec(w.shape, lambda i: (0, 0)),
            pl.BlockSpec((1, f2), lambda i: (0, 0)),
        ],
        out_specs=[
            pl.BlockSpec((R, f2), lambda i: (i, 0)),
            pl.BlockSpec((G, f2), lambda i: (0, 0)),
            pl.BlockSpec((G, f2), lambda i: (0, 0)),
        ],
        out_shape=[
            jax.ShapeDtypeStruct((N_PAD, f2), jnp.float32),
            jax.ShapeDtypeStruct((G, f2), jnp.float32),
            jax.ShapeDtypeStruct((G, f2), jnp.float32),
        ],
        scratch_shapes=[pltpu.VMEM((G, f2), jnp.float32),
                        pltpu.VMEM((G, f2), jnp.float32)],
        name=f"tc_matstats_f{fout}",
    )(agg, u, dinv, batch3, w, b2)


def _norm_body(fout, out_h, out_u, has_res, *refs):
    it = iter(refs)
    z_ref = next(it); s1_ref = next(it); s2_ref = next(it); cnt_ref = next(it)
    dinv_ref = next(it); batch_ref = next(it)
    gw_ref = next(it); gb_ref = next(it); gs_ref = next(it)
    res_ref = next(it) if has_res else None
    h_ref = next(it) if out_h else None
    u_ref = next(it) if out_u else None

    cnt = cnt_ref[...]
    ms = gs_ref[...]
    mean = s1_ref[...] / cnt
    var = s2_ref[...] / cnt - (2.0 - ms) * ms * mean * mean
    rstd = lax.rsqrt(var + EPS)
    mT = _onehotT(batch_ref)
    mrow = lax.dot_general(mT, mean, (((0,), (0,)), ((), ())),
                           preferred_element_type=jnp.float32,
                           precision=lax.Precision.HIGHEST)
    rrow = lax.dot_general(mT, rstd, (((0,), (0,)), ((), ())),
                           preferred_element_type=jnp.float32,
                           precision=lax.Precision.HIGHEST)
    z = z_ref[...]
    h = _gelu((z - mrow * ms) * rrow * gw_ref[...] + gb_ref[...])
    if has_res:
        h = h + res_ref[...]
    if out_h:
        h_ref[...] = h
    if out_u:
        dinv = dinv_ref[...]
        u_ref[...] = jnp.stack([h[:, :fout] * dinv, h[:, fout:] * dinv])


def _tc_norm(z, s1, s2, cnt, dinv, batch3, gw2, gb2, gs2, res, fout,
             out_h, out_u):
    grid = (NBLK,)
    f2 = 2 * fout
    in_specs = [
        pl.BlockSpec((R, f2), lambda i: (i, 0)),
        pl.BlockSpec((G, f2), lambda i: (0, 0)),
        pl.BlockSpec((G, f2), lambda i: (0, 0)),
        pl.BlockSpec((G, 1), lambda i: (0, 0)),
        pl.BlockSpec((R, 1), lambda i: (i, 0)),
        pl.BlockSpec((1, 1, R), lambda i: (i, 0, 0)),
        pl.BlockSpec((1, f2), lambda i: (0, 0)),
        pl.BlockSpec((1, f2), lambda i: (0, 0)),
        pl.BlockSpec((1, f2), lambda i: (0, 0)),
    ]
    args = [z, s1, s2, cnt, dinv, batch3, gw2, gb2, gs2]
    if res is not None:
        in_specs.append(pl.BlockSpec((R, f2), lambda i: (i, 0)))
        args.append(res)
    out_specs, out_shape = [], []
    if out_h:
        out_specs.append(pl.BlockSpec((R, f2), lambda i: (i, 0)))
        out_shape.append(jax.ShapeDtypeStruct((N_PAD, f2), jnp.float32))
    if out_u:
        out_specs.append(pl.BlockSpec((2, R, fout), lambda i: (0, i, 0)))
        out_shape.append(jax.ShapeDtypeStruct((2, N_PAD, fout), jnp.float32))
    res_out = pl.pallas_call(
        functools.partial(_norm_body, fout, out_h, out_u, res is not None),
        grid=grid,
        in_specs=in_specs,
        out_specs=out_specs,
        out_shape=out_shape,
        name=f"tc_norm_f{fout}",
    )(*args)
    return res_out


def kernel(x, batch, edge_index, W0, b0, W1, b1, W2, b2,
           gn0_w, gn0_b, gn0_s, gn1_w, gn1_b, gn1_s, gn2_w, gn2_b, gn2_s):
    f32 = jnp.float32
    i32 = jnp.int32

    src = edge_index[0].astype(i32)
    dst = edge_index[1].astype(i32)
    epad = E_PAD - E
    src_p = jnp.concatenate([src, jnp.zeros((epad,), i32)])
    dst_p = jnp.concatenate([dst, jnp.full((epad,), N_PAD - 1, i32)])
    srcA = jnp.stack([src_p, src_p + N_PAD]).reshape(2, ECHUNKS, 128)
    dstA = dst_p.reshape(ECHUNKS, 128)

    x2 = jnp.concatenate(
        [x.reshape(N, 2 * C), jnp.zeros((N_PAD - N, 2 * C), f32)])
    batch3 = jnp.concatenate(
        [batch.astype(i32), jnp.full((N_PAD - N,), -1, i32)]
    ).reshape(NBLK, 1, R)

    def two(v):
        return jnp.concatenate([v, v]).reshape(1, -1)

    b0t, b1t, b2t = two(b0), two(b1), two(b2)
    zrows16 = jnp.zeros((ZROWS, 16), f32)
    zrows32 = jnp.zeros((ZROWS, 32), f32)

    degp = _sc_degree(dstA)
    dinv, u0, cnt = _tc_prep(degp.T, x2, batch3)

    agg0 = _sc_aggregate(u0.reshape(2 * N_PAD, 16), srcA, dstA, zrows16, 16)
    z0, s1, s2 = _tc_matstats(agg0, u0, dinv, batch3, W0, b0t, 16, HID)
    h0, u1 = _tc_norm(z0, s1, s2, cnt, dinv, batch3, two(gn0_w), two(gn0_b),
                      two(gn0_s), None, HID, True, True)

    agg1 = _sc_aggregate(u1.reshape(2 * N_PAD, HID), srcA, dstA, zrows32, HID)
    z1, s1, s2 = _tc_matstats(agg1, u1, dinv, batch3, W1, b1t, HID, HID)
    (u2,) = _tc_norm(z1, s1, s2, cnt, dinv, batch3, two(gn1_w), two(gn1_b),
                     two(gn1_s), h0, HID, False, True)

    agg2 = _sc_aggregate(u2.reshape(2 * N_PAD, HID), srcA, dstA, zrows32, HID)
    z2, s1, s2 = _tc_matstats(agg2, u2, dinv, batch3, W2, b2t, HID, EMB)
    (h2,) = _tc_norm(z2, s1, s2, cnt, dinv, batch3, two(gn2_w), two(gn2_b),
                     two(gn2_s), None, EMB, True, False)

    x_out = h2[:N].reshape(N, 2, EMB)
    return (x_out, batch, edge_index)


# speedup vs baseline: 30.6375x; 1.4444x over previous
import functools

import jax
import jax.numpy as jnp
from jax import lax
from jax.experimental import pallas as pl
from jax.experimental.pallas import tpu as pltpu
from jax.experimental.pallas import tpu_sc as plsc

N = 50000
T = 2
C = 3
E = 800000
EMB = 128
HID = 32
G = 32

R = 1024
N_PAD = 50176
NBLK = N_PAD // R
ROWS_PER_TILE = N_PAD // 16

E_PAD = 802816
ECHUNKS = E_PAD // 128
CH_PER_TILE = ECHUNKS // 16
CH_PER_WORKER = ECHUNKS // 32
IGRP = 28
ZROWS = 448

EPS = 1e-5

_MESH = dict(core_axis_name="c", subcore_axis_name="s", num_cores=2,
             num_subcores=16)


def _deg_body(dst_hbm, out_hbm, hist, idxb):
    c = lax.axis_index("c")
    s = lax.axis_index("s")
    wid = s * 2 + c
    zero16 = jnp.zeros((16,), jnp.float32)
    one16 = jnp.ones((16,), jnp.float32)

    def zloop(i, _):
        hist[pl.ds(i * 16, 16)] = zero16
        return 0
    lax.fori_loop(0, N_PAD // 16, zloop, 0)

    pltpu.sync_copy(dst_hbm.at[pl.ds(wid * CH_PER_WORKER, CH_PER_WORKER)],
                    idxb)

    def eloop(g, _):
        for k in range(8):
            idx = idxb[g, pl.ds(k * 16, 16)]
            plsc.addupdate_scatter(hist, [idx], one16)
        return 0
    lax.fori_loop(0, CH_PER_WORKER, eloop, 0)

    pltpu.sync_copy(hist, out_hbm.at[wid])


def _sc_degree(dstA):
    mesh = plsc.VectorSubcoreMesh(**_MESH)
    return pl.kernel(
        _deg_body,
        out_type=jax.ShapeDtypeStruct((32, N_PAD), jnp.float32),
        mesh=mesh,
        scratch_types=[
            pltpu.VMEM((N_PAD,), jnp.float32),
            pltpu.VMEM((CH_PER_WORKER, 128), jnp.int32),
        ],
        compiler_params=pltpu.CompilerParams(use_tc_tiling_on_sc=False, needs_layout_passes=False),
        name="sc_degree",
    )(dstA)


NB = 2
NBURST_G = IGRP // NB


def _agg_body(tab_hbm, srcA_hbm, dstA_hbm, zrows_hbm, out_hbm,
              acc, srcb, dstb, rowsA, rowsB, semA, semB):
    c = lax.axis_index("c")
    s = lax.axis_index("s")

    def zloop(z, _):
        pltpu.sync_copy(zrows_hbm, acc.at[pl.ds(s * ROWS_PER_TILE + z * ZROWS,
                                                ZROWS)])
        return 0
    lax.fori_loop(0, ROWS_PER_TILE // ZROWS, zloop, 0)
    plsc.subcore_barrier()

    row0 = s * CH_PER_TILE

    def issue(burst, buf, sem):
        for b in range(NB):
            pltpu.async_copy(tab_hbm.at[srcb.at[burst * NB + b]],
                             buf.at[b], sem)

    def drain_scatter(burst, buf, sem):
        for b in range(NB):
            pltpu.make_async_copy(tab_hbm.at[pl.ds(0, 128)],
                                  buf.at[b], sem).wait()
        for b in range(NB):
            pltpu.sync_copy(buf.at[b], acc.at[dstb.at[burst * NB + b]],
                            add=True)

    def grp(g, _):
        pltpu.sync_copy(srcA_hbm.at[c, pl.ds(row0 + g * IGRP, IGRP)], srcb)
        pltpu.sync_copy(dstA_hbm.at[pl.ds(row0 + g * IGRP, IGRP)], dstb)
        issue(0, rowsA, semA)
        issue(1, rowsB, semB)

        def pair(k, _):
            drain_scatter(2 * k, rowsA, semA)

            @pl.when(2 * k + 2 < NBURST_G)
            def _():
                issue(2 * k + 2, rowsA, semA)
            drain_scatter(2 * k + 1, rowsB, semB)

            @pl.when(2 * k + 3 < NBURST_G)
            def _():
                issue(2 * k + 3, rowsB, semB)
            return 0
        lax.fori_loop(0, NBURST_G // 2, pair, 0)
        return 0
    lax.fori_loop(0, CH_PER_TILE // IGRP, grp, 0)

    plsc.subcore_barrier()
    sl = pl.ds(s * ROWS_PER_TILE, ROWS_PER_TILE)
    pltpu.sync_copy(acc.at[sl], out_hbm.at[c, sl])


def _sc_aggregate(tab, srcA, dstA, zrows, d):
    mesh = plsc.VectorSubcoreMesh(**_MESH)
    return pl.kernel(
        _agg_body,
        out_type=jax.ShapeDtypeStruct((2, N_PAD, d), jnp.float32),
        mesh=mesh,
        scratch_types=[
            pltpu.VMEM_SHARED((N_PAD, d), jnp.float32),
            pltpu.VMEM((IGRP, 128), jnp.int32),
            pltpu.VMEM((IGRP, 128), jnp.int32),
            pltpu.VMEM((NB, 128, d), jnp.float32),
            pltpu.VMEM((NB, 128, d), jnp.float32),
            pltpu.SemaphoreType.DMA,
            pltpu.SemaphoreType.DMA,
        ],
        compiler_params=pltpu.CompilerParams(use_tc_tiling_on_sc=False, needs_layout_passes=False),
        name=f"sc_aggregate_d{d}",
    )(tab, srcA, dstA, zrows)


def _onehotT(batch_ref):
    bb = batch_ref[0]
    gi = lax.broadcasted_iota(jnp.int32, (G, 1), 0)
    return (bb == gi).astype(jnp.float32)


def _gelu(v):
    return 0.5 * v * (1.0 + lax.erf(v * 0.7071067811865476))


def _prep_body(degp_ref, x2_ref, batch_ref, dinv_ref, u0_ref, cnt_ref,
               cnt_acc):
    i = pl.program_id(0)
    deg = jnp.sum(degp_ref[...], axis=1, keepdims=True) + 1.0
    dinv = lax.rsqrt(deg)
    dinv_ref[...] = dinv
    xb = x2_ref[...]
    pad = jnp.zeros((R, 16 - C), jnp.float32)
    u0 = jnp.stack([jnp.concatenate([xb[:, 0:C] * dinv, pad], axis=1),
                    jnp.concatenate([xb[:, C:2 * C] * dinv, pad], axis=1)])
    u0_ref[...] = u0
    mT = _onehotT(batch_ref)

    @pl.when(i == 0)
    def _():
        cnt_acc[...] = jnp.zeros_like(cnt_acc)
    cnt_acc[...] += jnp.sum(mT, axis=1, keepdims=True)

    @pl.when(i == NBLK - 1)
    def _():
        cnt_ref[...] = jnp.maximum(cnt_acc[...], 1.0)


def _tc_prep(degpT, x2, batch3):
    grid = (NBLK,)
    return pl.pallas_call(
        _prep_body,
        grid=grid,
        in_specs=[
            pl.BlockSpec((R, 32), lambda i: (i, 0)),
            pl.BlockSpec((R, 2 * C), lambda i: (i, 0)),
            pl.BlockSpec((1, 1, R), lambda i: (i, 0, 0)),
        ],
        out_specs=[
            pl.BlockSpec((R, 1), lambda i: (i, 0)),
            pl.BlockSpec((2, R, 16), lambda i: (0, i, 0)),
            pl.BlockSpec((G, 1), lambda i: (0, 0)),
        ],
        out_shape=[
            jax.ShapeDtypeStruct((N_PAD, 1), jnp.float32),
            jax.ShapeDtypeStruct((2, N_PAD, 16), jnp.float32),
            jax.ShapeDtypeStruct((G, 1), jnp.float32),
        ],
        scratch_shapes=[pltpu.VMEM((G, 1), jnp.float32)],
        name="tc_prep",
    )(degpT, x2, batch3)


def _matstats_body(din, fout, agg_ref, u_ref, dinv_ref, batch_ref, w_ref,
                   b_ref, z_ref, s1_ref, s2_ref, s1_acc, s2_acc):
    i = pl.program_id(0)
    dinv = dinv_ref[...]
    w = w_ref[...]
    zs = []
    for t in range(2):
        a = agg_ref[t] + u_ref[t]
        a = a[:, :w.shape[0]]
        zs.append(lax.dot_general(a, w, (((1,), (0,)), ((), ())),
                                  preferred_element_type=jnp.float32,
                           precision=lax.Precision.HIGHEST))
    z = jnp.concatenate(zs, axis=1) * dinv + b_ref[...]
    z_ref[...] = z
    mT = _onehotT(batch_ref)

    @pl.when(i == 0)
    def _():
        s1_acc[...] = jnp.zeros_like(s1_acc)
        s2_acc[...] = jnp.zeros_like(s2_acc)
    s1_acc[...] += lax.dot_general(mT, z, (((1,), (0,)), ((), ())),
                                   preferred_element_type=jnp.float32,
                           precision=lax.Precision.HIGHEST)
    s2_acc[...] += lax.dot_general(mT, z * z, (((1,), (0,)), ((), ())),
                                   preferred_element_type=jnp.float32,
                           precision=lax.Precision.HIGHEST)

    @pl.when(i == NBLK - 1)
    def _():
        s1_ref[...] = s1_acc[...]
        s2_ref[...] = s2_acc[...]


def _tc_matstats(agg, u, dinv, batch3, w, b2, din, fout):
    grid = (NBLK,)
    f2 = 2 * fout
    return pl.pallas_call(
        functools.partial(_matstats_body, din, fout),
        grid=grid,
        in_specs=[
            pl.BlockSpec((2, R, din), lambda i: (0, i, 0)),
            pl.BlockSpec((2, R, din), lambda i: (0, i, 0)),
            pl.BlockSpec((R, 1), lambda i: (i, 0)),
            pl.BlockSpec((1, 1, R), lambda i: (i, 0, 0)),
            pl.BlockSpec(w.shape, lambda i: (0, 0)),
            pl.BlockSpec((1, f2), lambda i: (0, 0)),
        ],
        out_specs=[
            pl.BlockSpec((R, f2), lambda i: (i, 0)),
            pl.BlockSpec((G, f2), lambda i: (0, 0)),
            pl.BlockSpec((G, f2), lambda i: (0, 0)),
        ],
        out_shape=[
            jax.ShapeDtypeStruct((N_PAD, f2), jnp.float32),
            jax.ShapeDtypeStruct((G, f2), jnp.float32),
            jax.ShapeDtypeStruct((G, f2), jnp.float32),
        ],
        scratch_shapes=[pltpu.VMEM((G, f2), jnp.float32),
                        pltpu.VMEM((G, f2), jnp.float32)],
        name=f"tc_matstats_f{fout}",
    )(agg, u, dinv, batch3, w, b2)


def _norm_body(fout, out_h, out_u, has_res, *refs):
    it = iter(refs)
    z_ref = next(it); s1_ref = next(it); s2_ref = next(it); cnt_ref = next(it)
    dinv_ref = next(it); batch_ref = next(it)
    gw_ref = next(it); gb_ref = next(it); gs_ref = next(it)
    res_ref = next(it) if has_res else None
    h_ref = next(it) if out_h else None
    u_ref = next(it) if out_u else None

    cnt = cnt_ref[...]
    ms = gs_ref[...]
    mean = s1_ref[...] / cnt
    var = s2_ref[...] / cnt - (2.0 - ms) * ms * mean * mean
    rstd = lax.rsqrt(var + EPS)
    mT = _onehotT(batch_ref)
    mrow = lax.dot_general(mT, mean, (((0,), (0,)), ((), ())),
                           preferred_element_type=jnp.float32,
                           precision=lax.Precision.HIGHEST)
    rrow = lax.dot_general(mT, rstd, (((0,), (0,)), ((), ())),
                           preferred_element_type=jnp.float32,
                           precision=lax.Precision.HIGHEST)
    z = z_ref[...]
    h = _gelu((z - mrow * ms) * rrow * gw_ref[...] + gb_ref[...])
    if has_res:
        h = h + res_ref[...]
    if out_h == "final":
        h_ref[:, 0, :] = h[:, :fout]
        h_ref[:, 1, :] = h[:, fout:]
    elif out_h:
        h_ref[...] = h
    if out_u:
        dinv = dinv_ref[...]
        u_ref[...] = jnp.stack([h[:, :fout] * dinv, h[:, fout:] * dinv])


def _tc_norm(z, s1, s2, cnt, dinv, batch3, gw2, gb2, gs2, res, fout,
             out_h, out_u):
    grid = (NBLK,)
    f2 = 2 * fout
    in_specs = [
        pl.BlockSpec((R, f2), lambda i: (i, 0)),
        pl.BlockSpec((G, f2), lambda i: (0, 0)),
        pl.BlockSpec((G, f2), lambda i: (0, 0)),
        pl.BlockSpec((G, 1), lambda i: (0, 0)),
        pl.BlockSpec((R, 1), lambda i: (i, 0)),
        pl.BlockSpec((1, 1, R), lambda i: (i, 0, 0)),
        pl.BlockSpec((1, f2), lambda i: (0, 0)),
        pl.BlockSpec((1, f2), lambda i: (0, 0)),
        pl.BlockSpec((1, f2), lambda i: (0, 0)),
    ]
    args = [z, s1, s2, cnt, dinv, batch3, gw2, gb2, gs2]
    if res is not None:
        in_specs.append(pl.BlockSpec((R, f2), lambda i: (i, 0)))
        args.append(res)
    out_specs, out_shape = [], []
    if out_h == "final":
        out_specs.append(pl.BlockSpec((R, 2, fout), lambda i: (i, 0, 0)))
        out_shape.append(jax.ShapeDtypeStruct((N, 2, fout), jnp.float32))
    elif out_h:
        out_specs.append(pl.BlockSpec((R, f2), lambda i: (i, 0)))
        out_shape.append(jax.ShapeDtypeStruct((N_PAD, f2), jnp.float32))
    if out_u:
        out_specs.append(pl.BlockSpec((2, R, fout), lambda i: (0, i, 0)))
        out_shape.append(jax.ShapeDtypeStruct((2, N_PAD, fout), jnp.float32))
    res_out = pl.pallas_call(
        functools.partial(_norm_body, fout, out_h, out_u, res is not None),
        grid=grid,
        in_specs=in_specs,
        out_specs=out_specs,
        out_shape=out_shape,
        name=f"tc_norm_f{fout}",
    )(*args)
    return res_out


def kernel(x, batch, edge_index, W0, b0, W1, b1, W2, b2,
           gn0_w, gn0_b, gn0_s, gn1_w, gn1_b, gn1_s, gn2_w, gn2_b, gn2_s):
    f32 = jnp.float32
    i32 = jnp.int32

    src = edge_index[0].astype(i32)
    dst = edge_index[1].astype(i32)
    epad = E_PAD - E
    src_p = jnp.concatenate([src, jnp.zeros((epad,), i32)])
    dst_p = jnp.concatenate([dst, jnp.full((epad,), N_PAD - 1, i32)])
    srcA = jnp.stack([src_p, src_p + N_PAD]).reshape(2, ECHUNKS, 128)
    dstA = dst_p.reshape(ECHUNKS, 128)

    x2 = jnp.concatenate(
        [x.reshape(N, 2 * C), jnp.zeros((N_PAD - N, 2 * C), f32)])
    batch3 = jnp.concatenate(
        [batch.astype(i32), jnp.full((N_PAD - N,), -1, i32)]
    ).reshape(NBLK, 1, R)

    def two(v):
        return jnp.concatenate([v, v]).reshape(1, -1)

    b0t, b1t, b2t = two(b0), two(b1), two(b2)
    zrows16 = jnp.zeros((ZROWS, 16), f32)
    zrows32 = jnp.zeros((ZROWS, 32), f32)

    degp = _sc_degree(dstA)
    dinv, u0, cnt = _tc_prep(degp.T, x2, batch3)

    agg0 = _sc_aggregate(u0.reshape(2 * N_PAD, 16), srcA, dstA, zrows16, 16)
    z0, s1, s2 = _tc_matstats(agg0, u0, dinv, batch3, W0, b0t, 16, HID)
    h0, u1 = _tc_norm(z0, s1, s2, cnt, dinv, batch3, two(gn0_w), two(gn0_b),
                      two(gn0_s), None, HID, True, True)

    agg1 = _sc_aggregate(u1.reshape(2 * N_PAD, HID), srcA, dstA, zrows32, HID)
    z1, s1, s2 = _tc_matstats(agg1, u1, dinv, batch3, W1, b1t, HID, HID)
    (u2,) = _tc_norm(z1, s1, s2, cnt, dinv, batch3, two(gn1_w), two(gn1_b),
                     two(gn1_s), h0, HID, False, True)

    agg2 = _sc_aggregate(u2.reshape(2 * N_PAD, HID), srcA, dstA, zrows32, HID)
    z2, s1, s2 = _tc_matstats(agg2, u2, dinv, batch3, W2, b2t, HID, EMB)
    (x_out,) = _tc_norm(z2, s1, s2, cnt, dinv, batch3, two(gn2_w), two(gn2_b),
                        two(gn2_s), None, EMB, "final", False)
    return (x_out, batch, edge_index)


# speedup vs baseline: 31.9887x; 1.0441x over previous
import functools

import jax
import jax.numpy as jnp
from jax import lax
from jax.experimental import pallas as pl
from jax.experimental.pallas import tpu as pltpu
from jax.experimental.pallas import tpu_sc as plsc

N = 50000
T = 2
C = 3
E = 800000
EMB = 128
HID = 32
G = 32

R = 2048
N_PAD = 51200
NBLK = N_PAD // R
ROWS_PER_TILE = N_PAD // 16

E_PAD = 802816
ECHUNKS = E_PAD // 128
CH_PER_TILE = ECHUNKS // 16
CH_PER_WORKER = ECHUNKS // 32
IGRP = 28
ZROWS = 640

EPS = 1e-5

_MESH = dict(core_axis_name="c", subcore_axis_name="s", num_cores=2,
             num_subcores=16)


def _deg_body(dst_hbm, out_hbm, hist, idxb):
    c = lax.axis_index("c")
    s = lax.axis_index("s")
    wid = s * 2 + c
    zero16 = jnp.zeros((16,), jnp.float32)
    one16 = jnp.ones((16,), jnp.float32)

    def zloop(i, _):
        hist[pl.ds(i * 16, 16)] = zero16
        return 0
    lax.fori_loop(0, N_PAD // 16, zloop, 0)

    pltpu.sync_copy(dst_hbm.at[pl.ds(wid * CH_PER_WORKER, CH_PER_WORKER)],
                    idxb)

    def eloop(g, _):
        for k in range(8):
            idx = idxb[g, pl.ds(k * 16, 16)]
            plsc.addupdate_scatter(hist, [idx], one16)
        return 0
    lax.fori_loop(0, CH_PER_WORKER, eloop, 0)

    pltpu.sync_copy(hist, out_hbm.at[wid])


def _sc_degree(dstA):
    mesh = plsc.VectorSubcoreMesh(**_MESH)
    return pl.kernel(
        _deg_body,
        out_type=jax.ShapeDtypeStruct((32, N_PAD), jnp.float32),
        mesh=mesh,
        scratch_types=[
            pltpu.VMEM((N_PAD,), jnp.float32),
            pltpu.VMEM((CH_PER_WORKER, 128), jnp.int32),
        ],
        compiler_params=pltpu.CompilerParams(use_tc_tiling_on_sc=False, needs_layout_passes=False),
        name="sc_degree",
    )(dstA)


NB = 2
NBURST_G = IGRP // NB


def _agg_body(tab_hbm, srcA_hbm, dstA_hbm, zrows_hbm, out_hbm,
              acc, srcb, dstb, rowsA, rowsB, semA, semB):
    c = lax.axis_index("c")
    s = lax.axis_index("s")

    def zloop(z, _):
        pltpu.sync_copy(zrows_hbm, acc.at[pl.ds(s * ROWS_PER_TILE + z * ZROWS,
                                                ZROWS)])
        return 0
    lax.fori_loop(0, ROWS_PER_TILE // ZROWS, zloop, 0)
    plsc.subcore_barrier()

    row0 = s * CH_PER_TILE

    def issue(burst, buf, sem):
        for b in range(NB):
            pltpu.async_copy(tab_hbm.at[srcb.at[burst * NB + b]],
                             buf.at[b], sem)

    def drain_scatter(burst, buf, sem):
        for b in range(NB):
            pltpu.make_async_copy(tab_hbm.at[pl.ds(0, 128)],
                                  buf.at[b], sem).wait()
        for b in range(NB):
            pltpu.sync_copy(buf.at[b], acc.at[dstb.at[burst * NB + b]],
                            add=True)

    def grp(g, _):
        pltpu.sync_copy(srcA_hbm.at[c, pl.ds(row0 + g * IGRP, IGRP)], srcb)
        pltpu.sync_copy(dstA_hbm.at[pl.ds(row0 + g * IGRP, IGRP)], dstb)
        issue(0, rowsA, semA)
        issue(1, rowsB, semB)

        def pair(k, _):
            drain_scatter(2 * k, rowsA, semA)

            @pl.when(2 * k + 2 < NBURST_G)
            def _():
                issue(2 * k + 2, rowsA, semA)
            drain_scatter(2 * k + 1, rowsB, semB)

            @pl.when(2 * k + 3 < NBURST_G)
            def _():
                issue(2 * k + 3, rowsB, semB)
            return 0
        lax.fori_loop(0, NBURST_G // 2, pair, 0)
        return 0
    lax.fori_loop(0, CH_PER_TILE // IGRP, grp, 0)

    plsc.subcore_barrier()
    sl = pl.ds(s * ROWS_PER_TILE, ROWS_PER_TILE)
    pltpu.sync_copy(acc.at[sl], out_hbm.at[c, sl])


def _sc_aggregate(tab, srcA, dstA, zrows, d):
    mesh = plsc.VectorSubcoreMesh(**_MESH)
    return pl.kernel(
        _agg_body,
        out_type=jax.ShapeDtypeStruct((2, N_PAD, d), jnp.float32),
        mesh=mesh,
        scratch_types=[
            pltpu.VMEM_SHARED((N_PAD, d), jnp.float32),
            pltpu.VMEM((IGRP, 128), jnp.int32),
            pltpu.VMEM((IGRP, 128), jnp.int32),
            pltpu.VMEM((NB, 128, d), jnp.float32),
            pltpu.VMEM((NB, 128, d), jnp.float32),
            pltpu.SemaphoreType.DMA,
            pltpu.SemaphoreType.DMA,
        ],
        compiler_params=pltpu.CompilerParams(use_tc_tiling_on_sc=False, needs_layout_passes=False),
        name=f"sc_aggregate_d{d}",
    )(tab, srcA, dstA, zrows)


def _onehotT(batch_ref):
    bb = batch_ref[0]
    gi = lax.broadcasted_iota(jnp.int32, (G, 1), 0)
    return (bb == gi).astype(jnp.float32)


def _gelu(v):
    return 0.5 * v * (1.0 + lax.erf(v * 0.7071067811865476))


def _prep_body(degp_ref, x2_ref, batch_ref, dinv_ref, u0_ref, cnt_ref,
               cnt_acc):
    i = pl.program_id(0)
    deg = jnp.sum(degp_ref[...], axis=1, keepdims=True) + 1.0
    dinv = lax.rsqrt(deg)
    dinv_ref[...] = dinv
    xb = x2_ref[...]
    pad = jnp.zeros((R, 16 - C), jnp.float32)
    u0 = jnp.stack([jnp.concatenate([xb[:, 0:C] * dinv, pad], axis=1),
                    jnp.concatenate([xb[:, C:2 * C] * dinv, pad], axis=1)])
    u0_ref[...] = u0
    mT = _onehotT(batch_ref)

    @pl.when(i == 0)
    def _():
        cnt_acc[...] = jnp.zeros_like(cnt_acc)
    cnt_acc[...] += jnp.sum(mT, axis=1, keepdims=True)

    @pl.when(i == NBLK - 1)
    def _():
        cnt_ref[...] = jnp.maximum(cnt_acc[...], 1.0)


def _tc_prep(degpT, x2, batch3):
    grid = (NBLK,)
    return pl.pallas_call(
        _prep_body,
        grid=grid,
        in_specs=[
            pl.BlockSpec((R, 32), lambda i: (i, 0)),
            pl.BlockSpec((R, 2 * C), lambda i: (i, 0)),
            pl.BlockSpec((1, 1, R), lambda i: (i, 0, 0)),
        ],
        out_specs=[
            pl.BlockSpec((R, 1), lambda i: (i, 0)),
            pl.BlockSpec((2, R, 16), lambda i: (0, i, 0)),
            pl.BlockSpec((G, 1), lambda i: (0, 0)),
        ],
        out_shape=[
            jax.ShapeDtypeStruct((N_PAD, 1), jnp.float32),
            jax.ShapeDtypeStruct((2, N_PAD, 16), jnp.float32),
            jax.ShapeDtypeStruct((G, 1), jnp.float32),
        ],
        scratch_shapes=[pltpu.VMEM((G, 1), jnp.float32)],
        name="tc_prep",
    )(degpT, x2, batch3)


def _matstats_body(din, fout, agg_ref, u_ref, dinv_ref, batch_ref, w_ref,
                   b_ref, z_ref, s1_ref, s2_ref, s1_acc, s2_acc):
    i = pl.program_id(0)
    dinv = dinv_ref[...]
    w = w_ref[...]
    zs = []
    for t in range(2):
        a = agg_ref[t] + u_ref[t]
        a = a[:, :w.shape[0]]
        zs.append(lax.dot_general(a, w, (((1,), (0,)), ((), ())),
                                  preferred_element_type=jnp.float32,
                           precision=lax.Precision.HIGHEST))
    z = jnp.concatenate(zs, axis=1) * dinv + b_ref[...]
    z_ref[...] = z
    mT = _onehotT(batch_ref)

    @pl.when(i == 0)
    def _():
        s1_acc[...] = jnp.zeros_like(s1_acc)
        s2_acc[...] = jnp.zeros_like(s2_acc)
    s1_acc[...] += lax.dot_general(mT, z, (((1,), (0,)), ((), ())),
                                   preferred_element_type=jnp.float32,
                           precision=lax.Precision.HIGHEST)
    s2_acc[...] += lax.dot_general(mT, z * z, (((1,), (0,)), ((), ())),
                                   preferred_element_type=jnp.float32,
                           precision=lax.Precision.HIGHEST)

    @pl.when(i == NBLK - 1)
    def _():
        s1_ref[...] = s1_acc[...]
        s2_ref[...] = s2_acc[...]


def _tc_matstats(agg, u, dinv, batch3, w, b2, din, fout):
    grid = (NBLK,)
    f2 = 2 * fout
    return pl.pallas_call(
        functools.partial(_matstats_body, din, fout),
        grid=grid,
        in_specs=[
            pl.BlockSpec((2, R, din), lambda i: (0, i, 0)),
            pl.BlockSpec((2, R, din), lambda i: (0, i, 0)),
            pl.BlockSpec((R, 1), lambda i: (i, 0)),
            pl.BlockSpec((1, 1, R), lambda i: (i, 0, 0)),
            pl.BlockSpec(w.shape, lambda i: (0, 0)),
            pl.BlockSpec((1, f2), lambda i: (0, 0)),
        ],
        out_specs=[
            pl.BlockSpec((R, f2), lambda i: (i, 0)),
            pl.BlockSpec((G, f2), lambda i: (0, 0)),
            pl.BlockSpec((G, f2), lambda i: (0, 0)),
        ],
        out_shape=[
            jax.ShapeDtypeStruct((N_PAD, f2), jnp.float32),
            jax.ShapeDtypeStruct((G, f2), jnp.float32),
            jax.ShapeDtypeStruct((G, f2), jnp.float32),
        ],
        scratch_shapes=[pltpu.VMEM((G, f2), jnp.float32),
                        pltpu.VMEM((G, f2), jnp.float32)],
        name=f"tc_matstats_f{fout}",
    )(agg, u, dinv, batch3, w, b2)


def _norm_body(fout, out_h, out_u, has_res, *refs):
    it = iter(refs)
    z_ref = next(it); s1_ref = next(it); s2_ref = next(it); cnt_ref = next(it)
    dinv_ref = next(it); batch_ref = next(it)
    gw_ref = next(it); gb_ref = next(it); gs_ref = next(it)
    res_ref = next(it) if has_res else None
    h_ref = next(it) if out_h else None
    u_ref = next(it) if out_u else None

    cnt = cnt_ref[...]
    ms = gs_ref[...]
    mean = s1_ref[...] / cnt
    var = s2_ref[...] / cnt - (2.0 - ms) * ms * mean * mean
    rstd = lax.rsqrt(var + EPS)
    mT = _onehotT(batch_ref)
    mrow = lax.dot_general(mT, mean, (((0,), (0,)), ((), ())),
                           preferred_element_type=jnp.float32,
                           precision=lax.Precision.HIGHEST)
    rrow = lax.dot_general(mT, rstd, (((0,), (0,)), ((), ())),
                           preferred_element_type=jnp.float32,
                           precision=lax.Precision.HIGHEST)
    z = z_ref[...]
    h = _gelu((z - mrow * ms) * rrow * gw_ref[...] + gb_ref[...])
    if has_res:
        h = h + res_ref[...]
    if out_h == "final":
        h_ref[:, 0, :] = h[:, :fout]
        h_ref[:, 1, :] = h[:, fout:]
    elif out_h:
        h_ref[...] = h
    if out_u:
        dinv = dinv_ref[...]
        u_ref[...] = jnp.stack([h[:, :fout] * dinv, h[:, fout:] * dinv])


def _tc_norm(z, s1, s2, cnt, dinv, batch3, gw2, gb2, gs2, res, fout,
             out_h, out_u):
    grid = (NBLK,)
    f2 = 2 * fout
    in_specs = [
        pl.BlockSpec((R, f2), lambda i: (i, 0)),
        pl.BlockSpec((G, f2), lambda i: (0, 0)),
        pl.BlockSpec((G, f2), lambda i: (0, 0)),
        pl.BlockSpec((G, 1), lambda i: (0, 0)),
        pl.BlockSpec((R, 1), lambda i: (i, 0)),
        pl.BlockSpec((1, 1, R), lambda i: (i, 0, 0)),
        pl.BlockSpec((1, f2), lambda i: (0, 0)),
        pl.BlockSpec((1, f2), lambda i: (0, 0)),
        pl.BlockSpec((1, f2), lambda i: (0, 0)),
    ]
    args = [z, s1, s2, cnt, dinv, batch3, gw2, gb2, gs2]
    if res is not None:
        in_specs.append(pl.BlockSpec((R, f2), lambda i: (i, 0)))
        args.append(res)
    out_specs, out_shape = [], []
    if out_h == "final":
        out_specs.append(pl.BlockSpec((R, 2, fout), lambda i: (i, 0, 0)))
        out_shape.append(jax.ShapeDtypeStruct((N, 2, fout), jnp.float32))
    elif out_h:
        out_specs.append(pl.BlockSpec((R, f2), lambda i: (i, 0)))
        out_shape.append(jax.ShapeDtypeStruct((N_PAD, f2), jnp.float32))
    if out_u:
        out_specs.append(pl.BlockSpec((2, R, fout), lambda i: (0, i, 0)))
        out_shape.append(jax.ShapeDtypeStruct((2, N_PAD, fout), jnp.float32))
    res_out = pl.pallas_call(
        functools.partial(_norm_body, fout, out_h, out_u, res is not None),
        grid=grid,
        in_specs=in_specs,
        out_specs=out_specs,
        out_shape=out_shape,
        name=f"tc_norm_f{fout}",
    )(*args)
    return res_out


def kernel(x, batch, edge_index, W0, b0, W1, b1, W2, b2,
           gn0_w, gn0_b, gn0_s, gn1_w, gn1_b, gn1_s, gn2_w, gn2_b, gn2_s):
    f32 = jnp.float32
    i32 = jnp.int32

    src = edge_index[0].astype(i32)
    dst = edge_index[1].astype(i32)
    epad = E_PAD - E
    src_p = jnp.concatenate([src, jnp.zeros((epad,), i32)])
    dst_p = jnp.concatenate([dst, jnp.full((epad,), N_PAD - 1, i32)])
    srcA = jnp.stack([src_p, src_p + N_PAD]).reshape(2, ECHUNKS, 128)
    dstA = dst_p.reshape(ECHUNKS, 128)

    x2 = jnp.concatenate(
        [x.reshape(N, 2 * C), jnp.zeros((N_PAD - N, 2 * C), f32)])
    batch3 = jnp.concatenate(
        [batch.astype(i32), jnp.full((N_PAD - N,), -1, i32)]
    ).reshape(NBLK, 1, R)

    def two(v):
        return jnp.concatenate([v, v]).reshape(1, -1)

    b0t, b1t, b2t = two(b0), two(b1), two(b2)
    zrows16 = jnp.zeros((ZROWS, 16), f32)
    zrows32 = jnp.zeros((ZROWS, 32), f32)

    degp = _sc_degree(dstA)
    dinv, u0, cnt = _tc_prep(degp.T, x2, batch3)

    agg0 = _sc_aggregate(u0.reshape(2 * N_PAD, 16), srcA, dstA, zrows16, 16)
    z0, s1, s2 = _tc_matstats(agg0, u0, dinv, batch3, W0, b0t, 16, HID)
    h0, u1 = _tc_norm(z0, s1, s2, cnt, dinv, batch3, two(gn0_w), two(gn0_b),
                      two(gn0_s), None, HID, True, True)

    agg1 = _sc_aggregate(u1.reshape(2 * N_PAD, HID), srcA, dstA, zrows32, HID)
    z1, s1, s2 = _tc_matstats(agg1, u1, dinv, batch3, W1, b1t, HID, HID)
    (u2,) = _tc_norm(z1, s1, s2, cnt, dinv, batch3, two(gn1_w), two(gn1_b),
                     two(gn1_s), h0, HID, False, True)

    agg2 = _sc_aggregate(u2.reshape(2 * N_PAD, HID), srcA, dstA, zrows32, HID)
    z2, s1, s2 = _tc_matstats(agg2, u2, dinv, batch3, W2, b2t, HID, EMB)
    (x_out,) = _tc_norm(z2, s1, s2, cnt, dinv, batch3, two(gn2_w), two(gn2_b),
                        two(gn2_s), None, EMB, "final", False)
    return (x_out, batch, edge_index)
